# Initial kernel scaffold; baseline (speedup 1.0000x reference)
import jax
import jax.numpy as jnp
from jax.experimental import pallas as pl


def kernel(x, input_np, output_np, edge_nt, edge_np, edge_sc, edge_index, in_np_table, out_np_table, edge_np_table, edge_nt_table, W1, b1, W2, b2, W_root, b_conv, W_sup, b_sup, W_nt, b_nt, W_tag, b_tag, W_prim, b_prim):
    raise NotImplementedError("write your pallas kernel here")


# speedup vs baseline: 1.8199x; 1.8199x over previous
import functools

import jax
import jax.numpy as jnp
from jax import lax
from jax.experimental import pallas as pl
from jax.experimental.pallas import tpu as pltpu
from jax.experimental.pallas import tpu_sc as plsc

N = 10000
E = 160000
F = 64
HID = 16

NC = 2
NS = 16
NW = NC * NS
EPW = E // NW
CH = 128
NFULL = EPW // CH
TAIL = EPW - NFULL * CH

NPAD = 10016
RPT = NPAD // NS

ET = 640
NT = 2000


def _node_prep_body(x_ref, inp_ref, outp_ref, tin_ref, tout_ref, o_ref):
    oh_in = (inp_ref[...] ==
             lax.broadcasted_iota(jnp.int32, (NT, 128), 1)).astype(jnp.float32)
    oh_out = (outp_ref[...] ==
              lax.broadcasted_iota(jnp.int32, (NT, 128), 1)).astype(jnp.float32)
    e_in = jnp.dot(oh_in, tin_ref[...], preferred_element_type=jnp.float32)
    e_out = jnp.dot(oh_out, tout_ref[...], preferred_element_type=jnp.float32)
    o_ref[...] = jnp.concatenate([x_ref[...], e_in, e_out], axis=1)


def _node_prep(x, input_np, output_np, tin_pad, tout_pad):
    return pl.pallas_call(
        _node_prep_body,
        grid=(N // NT,),
        in_specs=[
            pl.BlockSpec((NT, 32), lambda i: (i, 0)),
            pl.BlockSpec((NT, 1), lambda i: (i, 0)),
            pl.BlockSpec((NT, 1), lambda i: (i, 0)),
            pl.BlockSpec((128, 16), lambda i: (0, 0)),
            pl.BlockSpec((128, 16), lambda i: (0, 0)),
        ],
        out_specs=pl.BlockSpec((NT, F), lambda i: (i, 0)),
        out_shape=jax.ShapeDtypeStruct((N, F), jnp.float32),
    )(x, input_np.reshape(N, 1), output_np.reshape(N, 1), tin_pad, tout_pad)


def _gather_body(xcat_hbm, src_hbm, out_hbm, idx_v, rows_v, idx_t, rows_t, sem):
    wid = lax.axis_index("s") * NC + lax.axis_index("c")
    base = wid * EPW

    def chunk(c, _):
        off = base + c * CH
        pltpu.sync_copy(src_hbm.at[pl.ds(off, CH)], idx_v)
        pltpu.async_copy(xcat_hbm.at[idx_v], rows_v, sem).wait()
        pltpu.sync_copy(rows_v, out_hbm.at[pl.ds(off, CH)])
        return 0

    lax.fori_loop(0, NFULL, chunk, 0)
    off = base + NFULL * CH
    pltpu.sync_copy(src_hbm.at[pl.ds(off, TAIL)], idx_t)
    pltpu.async_copy(xcat_hbm.at[idx_t], rows_t, sem).wait()
    pltpu.sync_copy(rows_t, out_hbm.at[pl.ds(off, TAIL)])


def _sc_gather(x_cat, src):
    mesh = plsc.VectorSubcoreMesh(core_axis_name="c", subcore_axis_name="s")
    return pl.kernel(
        _gather_body,
        out_type=jax.ShapeDtypeStruct((E, F), jnp.float32),
        mesh=mesh,
        compiler_params=pltpu.CompilerParams(use_tc_tiling_on_sc=False),
        scratch_types=[
            pltpu.VMEM((CH,), jnp.int32),
            pltpu.VMEM((CH, F), jnp.float32),
            pltpu.VMEM((TAIL,), jnp.int32),
            pltpu.VMEM((TAIL, F), jnp.float32),
            pltpu.SemaphoreType.DMA,
        ],
    )(x_cat, src)


def _edge_body(nt_ref, np_ref, sc_ref, xj_ref, ant_ref, anp_ref, w1c_ref,
               b1_ref, w2p_ref, b2r_ref, o_ref):
    oh_nt = (nt_ref[...] ==
             lax.broadcasted_iota(jnp.int32, (ET, 16), 1)).astype(jnp.float32)
    oh_np = (np_ref[...] ==
             lax.broadcasted_iota(jnp.int32, (ET, 64), 1)).astype(jnp.float32)
    pre = (jnp.dot(oh_nt, ant_ref[...], preferred_element_type=jnp.float32)
           + jnp.dot(oh_np, anp_ref[...], preferred_element_type=jnp.float32)
           + sc_ref[...] * w1c_ref[...] + b1_ref[...])
    h = jnp.maximum(pre, 0.0)
    xj = xj_ref[...]
    h2p = jnp.dot(h, w2p_ref[...], preferred_element_type=jnp.float32)
    msg = jnp.dot(xj, b2r_ref[...], preferred_element_type=jnp.float32)
    cols = [jnp.sum(h2p[:, o * 64:(o + 1) * 64] * xj, axis=1, keepdims=True)
            for o in range(HID)]
    msg = msg + jnp.concatenate(cols, axis=1)
    cnt_cols = (lax.broadcasted_iota(jnp.int32, (ET, 16), 1) == 0
                ).astype(jnp.float32)
    o_ref[...] = jnp.concatenate([msg, cnt_cols], axis=1)


def _edge_stage(edge_nt, edge_np, edge_sc, x_j, A_nt, A_np, w1c, b1, W2p, B2r):
    return pl.pallas_call(
        _edge_body,
        grid=(E // ET,),
        in_specs=[
            pl.BlockSpec((ET, 1), lambda i: (i, 0)),
            pl.BlockSpec((ET, 1), lambda i: (i, 0)),
            pl.BlockSpec((ET, 1), lambda i: (i, 0)),
            pl.BlockSpec((ET, F), lambda i: (i, 0)),
            pl.BlockSpec((16, 64), lambda i: (0, 0)),
            pl.BlockSpec((64, 64), lambda i: (0, 0)),
            pl.BlockSpec((1, 64), lambda i: (0, 0)),
            pl.BlockSpec((1, 64), lambda i: (0, 0)),
            pl.BlockSpec((64, 1024), lambda i: (0, 0)),
            pl.BlockSpec((64, 16), lambda i: (0, 0)),
        ],
        out_specs=pl.BlockSpec((ET, 32), lambda i: (i, 0)),
        out_shape=jax.ShapeDtypeStruct((E, 32), jnp.float32),
    )(edge_nt.reshape(E, 1), edge_np.reshape(E, 1), edge_sc.reshape(E, 1),
      x_j, A_nt, A_np, w1c, b1, W2p, B2r)


def _scatter_body(msg_hbm, dst_hbm, zeros_hbm, out_hbm,
                  idx_v, val_v, idx_t, val_t, acc):
    c = lax.axis_index("c")
    s = lax.axis_index("s")
    wid = s * NC + c
    pltpu.sync_copy(zeros_hbm, acc.at[pl.ds(s * RPT, RPT)])
    plsc.subcore_barrier()

    base = wid * EPW

    def chunk(k, _):
        off = base + k * CH
        pltpu.sync_copy(dst_hbm.at[pl.ds(off, CH)], idx_v)
        pltpu.sync_copy(msg_hbm.at[pl.ds(off, CH)], val_v)
        pltpu.sync_copy(val_v, acc.at[idx_v], add=True)
        return 0

    lax.fori_loop(0, NFULL, chunk, 0)
    off = base + NFULL * CH
    pltpu.sync_copy(dst_hbm.at[pl.ds(off, TAIL)], idx_t)
    pltpu.sync_copy(msg_hbm.at[pl.ds(off, TAIL)], val_t)
    pltpu.sync_copy(val_t, acc.at[idx_t], add=True)

    plsc.subcore_barrier()
    pltpu.sync_copy(acc.at[pl.ds(s * RPT, RPT)],
                    out_hbm.at[c, pl.ds(s * RPT, RPT)])


def _sc_scatter(msg_aug, dst, zeros_blk):
    mesh = plsc.VectorSubcoreMesh(core_axis_name="c", subcore_axis_name="s")
    return pl.kernel(
        _scatter_body,
        out_type=jax.ShapeDtypeStruct((NC, NPAD, 32), jnp.float32),
        mesh=mesh,
        compiler_params=pltpu.CompilerParams(use_tc_tiling_on_sc=False),
        scratch_types=[
            pltpu.VMEM((CH,), jnp.int32),
            pltpu.VMEM((CH, 32), jnp.float32),
            pltpu.VMEM((TAIL,), jnp.int32),
            pltpu.VMEM((TAIL, 32), jnp.float32),
            pltpu.VMEM_SHARED((NPAD, 32), jnp.float32),
        ],
    )(msg_aug, dst, zeros_blk)


def _final_body(p0_ref, p1_ref, xcat_ref, wroot_ref, bconv_ref, wh_ref,
                bh_ref, o_ref):
    s = p0_ref[...] + p1_ref[...]
    sums = s[:, :HID]
    cnt = s[:, HID:HID + 1]
    agg = sums / jnp.maximum(cnt, 1.0)
    root = jnp.dot(xcat_ref[...], wroot_ref[...],
                   preferred_element_type=jnp.float32)
    hnode = jnp.maximum(agg + root + bconv_ref[...], 0.0)
    o_ref[...] = jnp.dot(hnode, wh_ref[...],
                         preferred_element_type=jnp.float32) + bh_ref[...]


def _final_stage(p0, p1, x_cat, W_root, b_conv, Wh, bh):
    return pl.pallas_call(
        _final_body,
        grid=(N // NT,),
        in_specs=[
            pl.BlockSpec((NT, 32), lambda i: (i, 0)),
            pl.BlockSpec((NT, 32), lambda i: (i, 0)),
            pl.BlockSpec((NT, F), lambda i: (i, 0)),
            pl.BlockSpec((F, HID), lambda i: (0, 0)),
            pl.BlockSpec((1, HID), lambda i: (0, 0)),
            pl.BlockSpec((HID, 128), lambda i: (0, 0)),
            pl.BlockSpec((1, 128), lambda i: (0, 0)),
        ],
        out_specs=pl.BlockSpec((NT, 128), lambda i: (i, 0)),
        out_shape=jax.ShapeDtypeStruct((N, 128), jnp.float32),
    )(p0, p1, x_cat, W_root, b_conv, Wh, bh)


def kernel(x, input_np, output_np, edge_nt, edge_np, edge_sc, edge_index,
           in_np_table, out_np_table, edge_np_table, edge_nt_table,
           W1, b1, W2, b2, W_root, b_conv,
           W_sup, b_sup, W_nt, b_nt, W_tag, b_tag, W_prim, b_prim):
    f32 = jnp.float32
    A_nt = jnp.zeros((16, 64), f32).at[:10].set(edge_nt_table @ W1[:16])
    A_np = jnp.zeros((64, 64), f32).at[:50].set(edge_np_table @ W1[16:32])
    w1c = W1[32].reshape(1, 64)
    b1r = b1.reshape(1, 64)
    W2p = W2.reshape(64, 64, HID).transpose(0, 2, 1).reshape(64, 64 * HID)
    B2r = b2.reshape(64, HID)
    tin_pad = jnp.zeros((128, 16), f32).at[:100].set(in_np_table)
    tout_pad = jnp.zeros((128, 16), f32).at[:100].set(out_np_table)
    Wh = jnp.zeros((HID, 128), f32)
    Wh = Wh.at[:, 0:16].set(W_sup).at[:, 16:26].set(W_nt)
    Wh = Wh.at[:, 26:90].set(W_tag).at[:, 90:122].set(W_prim)
    bh = jnp.zeros((1, 128), f32)
    bh = bh.at[0, 0:16].set(b_sup).at[0, 16:26].set(b_nt)
    bh = bh.at[0, 26:90].set(b_tag).at[0, 90:122].set(b_prim)

    src = edge_index[0].astype(jnp.int32)
    dst = edge_index[1].astype(jnp.int32)

    x_cat = _node_prep(x, input_np.astype(jnp.int32),
                       output_np.astype(jnp.int32), tin_pad, tout_pad)
    x_j = _sc_gather(x_cat, src)
    msg_aug = _edge_stage(edge_nt.astype(jnp.int32), edge_np.astype(jnp.int32),
                          edge_sc, x_j, A_nt, A_np, w1c, b1r, W2p, B2r)
    zeros_blk = jnp.zeros((RPT, 32), f32)
    partials = _sc_scatter(msg_aug, dst, zeros_blk)
    out = _final_stage(partials[0], partials[1], x_cat, W_root,
                       b_conv.reshape(1, HID), Wh, bh)
    return (out[:, 0:16], out[:, 16:26], out[:, 26:90], out[:, 90:122])


# speedup vs baseline: 3.1165x; 1.7124x over previous
import functools

import jax
import jax.numpy as jnp
from jax import lax
from jax.experimental import pallas as pl
from jax.experimental.pallas import tpu as pltpu
from jax.experimental.pallas import tpu_sc as plsc

N = 10000
E = 160000
F = 64
HID = 16

NC = 2
NS = 16
NW = NC * NS
EPW = E // NW
CH = 128
NFULL = EPW // CH
TAIL = EPW - NFULL * CH

NPAD = 10016
RPT = NPAD // NS

ET = 640
NT = 2000


def _node_prep_body(x_ref, inp_ref, outp_ref, tin_ref, tout_ref, o_ref):
    oh_in = (inp_ref[...] ==
             lax.broadcasted_iota(jnp.int32, (NT, 128), 1)).astype(jnp.float32)
    oh_out = (outp_ref[...] ==
              lax.broadcasted_iota(jnp.int32, (NT, 128), 1)).astype(jnp.float32)
    e_in = jnp.dot(oh_in, tin_ref[...], preferred_element_type=jnp.float32)
    e_out = jnp.dot(oh_out, tout_ref[...], preferred_element_type=jnp.float32)
    o_ref[...] = jnp.concatenate([x_ref[...], e_in, e_out], axis=1)


def _node_prep(x, input_np, output_np, tin_pad, tout_pad):
    return pl.pallas_call(
        _node_prep_body,
        grid=(N // NT,),
        in_specs=[
            pl.BlockSpec((NT, 32), lambda i: (i, 0)),
            pl.BlockSpec((NT, 1), lambda i: (i, 0)),
            pl.BlockSpec((NT, 1), lambda i: (i, 0)),
            pl.BlockSpec((128, 16), lambda i: (0, 0)),
            pl.BlockSpec((128, 16), lambda i: (0, 0)),
        ],
        out_specs=pl.BlockSpec((NT, F), lambda i: (i, 0)),
        out_shape=jax.ShapeDtypeStruct((N, F), jnp.float32),
    )(x, input_np.reshape(N, 1), output_np.reshape(N, 1), tin_pad, tout_pad)


def _gather_body(xcat_hbm, src_hbm, out_hbm, idx_v, rows_v, idx_t, rows_t, sem):
    wid = lax.axis_index("s") * NC + lax.axis_index("c")
    base = wid * EPW

    def chunk(c, _):
        off = base + c * CH
        pltpu.sync_copy(src_hbm.at[pl.ds(off, CH)], idx_v)
        pltpu.async_copy(xcat_hbm.at[idx_v], rows_v, sem).wait()
        pltpu.sync_copy(rows_v, out_hbm.at[pl.ds(off, CH)])
        return 0

    lax.fori_loop(0, NFULL, chunk, 0)
    off = base + NFULL * CH
    pltpu.sync_copy(src_hbm.at[pl.ds(off, TAIL)], idx_t)
    pltpu.async_copy(xcat_hbm.at[idx_t], rows_t, sem).wait()
    pltpu.sync_copy(rows_t, out_hbm.at[pl.ds(off, TAIL)])


def _sc_gather(x_cat, src):
    mesh = plsc.VectorSubcoreMesh(core_axis_name="c", subcore_axis_name="s")
    return pl.kernel(
        _gather_body,
        out_type=jax.ShapeDtypeStruct((E, F), jnp.float32),
        mesh=mesh,
        compiler_params=pltpu.CompilerParams(use_tc_tiling_on_sc=False),
        scratch_types=[
            pltpu.VMEM((CH,), jnp.int32),
            pltpu.VMEM((CH, F), jnp.float32),
            pltpu.VMEM((TAIL,), jnp.int32),
            pltpu.VMEM((TAIL, F), jnp.float32),
            pltpu.SemaphoreType.DMA,
        ],
    )(x_cat, src)


def _edge_body(nt_ref, np_ref, sc_ref, xj_ref, ant_ref, anp_ref, w1c_ref,
               b1_ref, w2p_ref, b2r_ref, sel_ref, o_ref):
    oh_nt = (nt_ref[...] ==
             lax.broadcasted_iota(jnp.int32, (ET, 16), 1)).astype(jnp.float32)
    oh_np = (np_ref[...] ==
             lax.broadcasted_iota(jnp.int32, (ET, 64), 1)).astype(jnp.float32)
    pre = (jnp.dot(oh_nt, ant_ref[...], preferred_element_type=jnp.float32)
           + jnp.dot(oh_np, anp_ref[...], preferred_element_type=jnp.float32)
           + sc_ref[...] * w1c_ref[...] + b1_ref[...])
    h = jnp.maximum(pre, 0.0)
    xj = xj_ref[...]
    h2p = jnp.dot(h, w2p_ref[...], preferred_element_type=jnp.float32)
    msg = jnp.dot(xj, b2r_ref[...], preferred_element_type=jnp.float32)
    prod = h2p * jnp.tile(xj, (1, HID))
    msg = msg + jnp.dot(prod, sel_ref[...], preferred_element_type=jnp.float32)
    cnt_cols = (lax.broadcasted_iota(jnp.int32, (ET, 16), 1) == 0
                ).astype(jnp.float32)
    o_ref[...] = jnp.concatenate([msg, cnt_cols], axis=1)


def _edge_stage(edge_nt, edge_np, edge_sc, x_j, A_nt, A_np, w1c, b1, W2p, B2r,
                Sel):
    return pl.pallas_call(
        _edge_body,
        grid=(E // ET,),
        in_specs=[
            pl.BlockSpec((ET, 1), lambda i: (i, 0)),
            pl.BlockSpec((ET, 1), lambda i: (i, 0)),
            pl.BlockSpec((ET, 1), lambda i: (i, 0)),
            pl.BlockSpec((ET, F), lambda i: (i, 0)),
            pl.BlockSpec((16, 64), lambda i: (0, 0)),
            pl.BlockSpec((64, 64), lambda i: (0, 0)),
            pl.BlockSpec((1, 64), lambda i: (0, 0)),
            pl.BlockSpec((1, 64), lambda i: (0, 0)),
            pl.BlockSpec((64, 1024), lambda i: (0, 0)),
            pl.BlockSpec((64, 16), lambda i: (0, 0)),
            pl.BlockSpec((1024, 16), lambda i: (0, 0)),
        ],
        out_specs=pl.BlockSpec((ET, 32), lambda i: (i, 0)),
        out_shape=jax.ShapeDtypeStruct((E, 32), jnp.float32),
    )(edge_nt.reshape(E, 1), edge_np.reshape(E, 1), edge_sc.reshape(E, 1),
      x_j, A_nt, A_np, w1c, b1, W2p, B2r, Sel)


def _scatter_body(msg_hbm, dst_hbm, zeros_hbm, out_hbm,
                  idx_v, val_v, idx_t, val_t, acc):
    c = lax.axis_index("c")
    s = lax.axis_index("s")
    wid = s * NC + c
    pltpu.sync_copy(zeros_hbm, acc.at[pl.ds(s * RPT, RPT)])
    plsc.subcore_barrier()

    base = wid * EPW

    def chunk(k, _):
        off = base + k * CH
        pltpu.sync_copy(dst_hbm.at[pl.ds(off, CH)], idx_v)
        pltpu.sync_copy(msg_hbm.at[pl.ds(off, CH)], val_v)
        pltpu.sync_copy(val_v, acc.at[idx_v], add=True)
        return 0

    lax.fori_loop(0, NFULL, chunk, 0)
    off = base + NFULL * CH
    pltpu.sync_copy(dst_hbm.at[pl.ds(off, TAIL)], idx_t)
    pltpu.sync_copy(msg_hbm.at[pl.ds(off, TAIL)], val_t)
    pltpu.sync_copy(val_t, acc.at[idx_t], add=True)

    plsc.subcore_barrier()
    pltpu.sync_copy(acc.at[pl.ds(s * RPT, RPT)],
                    out_hbm.at[c, pl.ds(s * RPT, RPT)])


def _sc_scatter(msg_aug, dst, zeros_blk):
    mesh = plsc.VectorSubcoreMesh(core_axis_name="c", subcore_axis_name="s")
    return pl.kernel(
        _scatter_body,
        out_type=jax.ShapeDtypeStruct((NC, NPAD, 32), jnp.float32),
        mesh=mesh,
        compiler_params=pltpu.CompilerParams(use_tc_tiling_on_sc=False),
        scratch_types=[
            pltpu.VMEM((CH,), jnp.int32),
            pltpu.VMEM((CH, 32), jnp.float32),
            pltpu.VMEM((TAIL,), jnp.int32),
            pltpu.VMEM((TAIL, 32), jnp.float32),
            pltpu.VMEM_SHARED((NPAD, 32), jnp.float32),
        ],
    )(msg_aug, dst, zeros_blk)


def _final_body(p0_ref, p1_ref, xcat_ref, wroot_ref, bconv_ref, wh_ref,
                bh_ref, o_ref):
    s = p0_ref[...] + p1_ref[...]
    sums = s[:, :HID]
    cnt = s[:, HID:HID + 1]
    agg = sums / jnp.maximum(cnt, 1.0)
    root = jnp.dot(xcat_ref[...], wroot_ref[...],
                   preferred_element_type=jnp.float32)
    hnode = jnp.maximum(agg + root + bconv_ref[...], 0.0)
    o_ref[...] = jnp.dot(hnode, wh_ref[...],
                         preferred_element_type=jnp.float32) + bh_ref[...]


def _final_stage(p0, p1, x_cat, W_root, b_conv, Wh, bh):
    return pl.pallas_call(
        _final_body,
        grid=(N // NT,),
        in_specs=[
            pl.BlockSpec((NT, 32), lambda i: (i, 0)),
            pl.BlockSpec((NT, 32), lambda i: (i, 0)),
            pl.BlockSpec((NT, F), lambda i: (i, 0)),
            pl.BlockSpec((F, HID), lambda i: (0, 0)),
            pl.BlockSpec((1, HID), lambda i: (0, 0)),
            pl.BlockSpec((HID, 128), lambda i: (0, 0)),
            pl.BlockSpec((1, 128), lambda i: (0, 0)),
        ],
        out_specs=pl.BlockSpec((NT, 128), lambda i: (i, 0)),
        out_shape=jax.ShapeDtypeStruct((N, 128), jnp.float32),
    )(p0, p1, x_cat, W_root, b_conv, Wh, bh)


def kernel(x, input_np, output_np, edge_nt, edge_np, edge_sc, edge_index,
           in_np_table, out_np_table, edge_np_table, edge_nt_table,
           W1, b1, W2, b2, W_root, b_conv,
           W_sup, b_sup, W_nt, b_nt, W_tag, b_tag, W_prim, b_prim):
    f32 = jnp.float32
    A_nt = jnp.zeros((16, 64), f32).at[:10].set(edge_nt_table @ W1[:16])
    A_np = jnp.zeros((64, 64), f32).at[:50].set(edge_np_table @ W1[16:32])
    w1c = W1[32].reshape(1, 64)
    b1r = b1.reshape(1, 64)
    W2p = W2.reshape(64, 64, HID).transpose(0, 2, 1).reshape(64, 64 * HID)
    B2r = b2.reshape(64, HID)
    Sel = (jnp.arange(64 * HID)[:, None] // 64
           == jnp.arange(HID)[None, :]).astype(f32)
    tin_pad = jnp.zeros((128, 16), f32).at[:100].set(in_np_table)
    tout_pad = jnp.zeros((128, 16), f32).at[:100].set(out_np_table)
    Wh = jnp.zeros((HID, 128), f32)
    Wh = Wh.at[:, 0:16].set(W_sup).at[:, 16:26].set(W_nt)
    Wh = Wh.at[:, 26:90].set(W_tag).at[:, 90:122].set(W_prim)
    bh = jnp.zeros((1, 128), f32)
    bh = bh.at[0, 0:16].set(b_sup).at[0, 16:26].set(b_nt)
    bh = bh.at[0, 26:90].set(b_tag).at[0, 90:122].set(b_prim)

    src = edge_index[0].astype(jnp.int32)
    dst = edge_index[1].astype(jnp.int32)

    x_cat = _node_prep(x, input_np.astype(jnp.int32),
                       output_np.astype(jnp.int32), tin_pad, tout_pad)
    x_j = _sc_gather(x_cat, src)
    msg_aug = _edge_stage(edge_nt.astype(jnp.int32), edge_np.astype(jnp.int32),
                          edge_sc, x_j, A_nt, A_np, w1c, b1r, W2p, B2r, Sel)
    zeros_blk = jnp.zeros((RPT, 32), f32)
    partials = _sc_scatter(msg_aug, dst, zeros_blk)
    out = _final_stage(partials[0], partials[1], x_cat, W_root,
                       b_conv.reshape(1, HID), Wh, bh)
    return (out[:, 0:16], out[:, 16:26], out[:, 26:90], out[:, 90:122])
